# Initial kernel scaffold; baseline (speedup 1.0000x reference)
import jax
import jax.numpy as jnp
from jax.experimental import pallas as pl


def kernel(x, edge_index, W, b):
    raise NotImplementedError("write your pallas kernel here")


# speedup vs baseline: 15.8888x; 15.8888x over previous
import functools

import jax
import jax.numpy as jnp
from jax import lax
from jax.experimental import pallas as pl
from jax.experimental.pallas import tpu as pltpu
from jax.experimental.pallas import tpu_sc as plsc

_LANES = 16


def _histogram_sc(col, n_nodes):
    info = plsc.get_sparse_core_info()
    nc, ns = info.num_cores, info.num_subcores
    nw = nc * ns
    e = col.shape[0]
    e_per_w = e // nw
    assert e % nw == 0 and e_per_w % _LANES == 0 and n_nodes % _LANES == 0

    mesh = plsc.VectorSubcoreMesh(core_axis_name="c", subcore_axis_name="s")

    @functools.partial(
        pl.kernel,
        mesh=mesh,
        out_type=jax.ShapeDtypeStruct((nw, n_nodes), jnp.float32),
        scratch_types=[
            pltpu.VMEM((e_per_w,), jnp.int32),
            pltpu.VMEM((n_nodes,), jnp.float32),
        ],
        compiler_params=pltpu.CompilerParams(needs_layout_passes=False),
    )
    def hist_kernel(col_hbm, out_hbm, idx_v, hist_v):
        wid = lax.axis_index("s") * nc + lax.axis_index("c")
        base = wid * e_per_w
        pltpu.sync_copy(col_hbm.at[pl.ds(base, e_per_w)], idx_v)

        zeros = jnp.zeros((_LANES,), jnp.float32)

        def zero_body(i, carry):
            hist_v[pl.ds(i * _LANES, _LANES)] = zeros
            return carry

        lax.fori_loop(0, n_nodes // _LANES, zero_body, 0)

        ones = jnp.ones((_LANES,), jnp.float32)

        def scat_body(i, carry):
            idx = idx_v[pl.ds(i * _LANES, _LANES)]
            plsc.addupdate_scatter(hist_v, [idx], ones)
            return carry

        lax.fori_loop(0, e_per_w // _LANES, scat_body, 0)

        pltpu.sync_copy(hist_v, out_hbm.at[wid])

    return hist_kernel(col)


def _fused_tc(partials, x, w, b2d, inv_e):
    n, c_in = x.shape
    c_out = w.shape[0]

    def body(p_ref, x_ref, w_ref, b_ref, out_ref):
        xv = x_ref[...]
        w1 = w_ref[:, :c_in]
        w2 = w_ref[:, c_in:]
        pm = lax.dot_general(p_ref[...], xv, (((1,), (0,)), ((), ())),
                             preferred_element_type=jnp.float32)
        m = jnp.sum(pm, axis=0, keepdims=True) * inv_e
        m8 = jnp.broadcast_to(m, (8, c_in))
        const8 = lax.dot_general(m8, w2, (((1,), (1,)), ((), ())),
                                 preferred_element_type=jnp.float32)
        const = const8[0:1, :] + b_ref[...]
        y = lax.dot_general(xv, w1, (((1,), (1,)), ((), ())),
                            preferred_element_type=jnp.float32)
        out_ref[...] = y + const

    return pl.pallas_call(
        body,
        out_shape=jax.ShapeDtypeStruct((n, c_out), jnp.float32),
    )(partials, x, w, b2d)


def kernel(x, edge_index, W, b):
    n, _ = x.shape
    e = edge_index.shape[1]
    col = edge_index[1].astype(jnp.int32)
    partials = _histogram_sc(col, n)
    return _fused_tc(partials, x, W, b.reshape(1, -1), 1.0 / e)


# speedup vs baseline: 20.4350x; 1.2861x over previous
import functools

import jax
import jax.numpy as jnp
from jax import lax
from jax.experimental import pallas as pl
from jax.experimental.pallas import tpu as pltpu
from jax.experimental.pallas import tpu_sc as plsc

_LANES = 16


def _histogram_sc(ei_flat, n_nodes):
    info = plsc.get_sparse_core_info()
    nc, ns = info.num_cores, info.num_subcores
    nw = nc * ns
    e = ei_flat.shape[0] // 2
    e_per_w = e // nw
    assert e % nw == 0 and e_per_w % _LANES == 0 and n_nodes % _LANES == 0

    mesh = plsc.VectorSubcoreMesh(core_axis_name="c", subcore_axis_name="s")

    @functools.partial(
        pl.kernel,
        mesh=mesh,
        out_type=jax.ShapeDtypeStruct((nw, n_nodes), jnp.float32),
        scratch_types=[
            pltpu.VMEM((e_per_w,), jnp.int32),
            pltpu.VMEM((n_nodes,), jnp.float32),
        ],
        compiler_params=pltpu.CompilerParams(needs_layout_passes=False),
    )
    def hist_kernel(ei_hbm, out_hbm, idx_v, hist_v):
        wid = lax.axis_index("s") * nc + lax.axis_index("c")
        base = e + wid * e_per_w
        pltpu.sync_copy(ei_hbm.at[pl.ds(base, e_per_w)], idx_v)

        zeros = jnp.zeros((_LANES,), jnp.float32)

        def zero_body(i, carry):
            hist_v[pl.ds(i * _LANES, _LANES)] = zeros
            return carry

        lax.fori_loop(0, n_nodes // _LANES, zero_body, 0)

        ones = jnp.ones((_LANES,), jnp.float32)

        def scat_body(i, carry):
            idx = idx_v[pl.ds(i * _LANES, _LANES)]
            plsc.addupdate_scatter(hist_v, [idx], ones)
            return carry

        lax.fori_loop(0, e_per_w // _LANES, scat_body, 0)

        pltpu.sync_copy(hist_v, out_hbm.at[wid])

    return hist_kernel(ei_flat)


def _fused_tc(partials, x, w, b2d, inv_e):
    n, c_in = x.shape
    c_out = w.shape[0]

    def body(p_ref, x_ref, w_ref, b_ref, out_ref):
        xv = x_ref[...]
        w1 = w_ref[:, :c_in]
        w2 = w_ref[:, c_in:]
        pm = lax.dot_general(p_ref[...], xv, (((1,), (0,)), ((), ())),
                             preferred_element_type=jnp.float32)
        m = jnp.sum(pm, axis=0, keepdims=True) * inv_e
        m8 = jnp.broadcast_to(m, (8, c_in))
        const8 = lax.dot_general(m8, w2, (((1,), (1,)), ((), ())),
                                 preferred_element_type=jnp.float32)
        const = const8[0:1, :] + b_ref[...]
        y = lax.dot_general(xv, w1, (((1,), (1,)), ((), ())),
                            preferred_element_type=jnp.float32)
        out_ref[...] = y + const

    return pl.pallas_call(
        body,
        out_shape=jax.ShapeDtypeStruct((n, c_out), jnp.float32),
    )(partials, x, w, b2d)


def kernel(x, edge_index, W, b):
    n, _ = x.shape
    e = edge_index.shape[1]
    partials = _histogram_sc(edge_index.astype(jnp.int32).reshape(-1), n)
    return _fused_tc(partials, x, W, b.reshape(1, -1), 1.0 / e)


# speedup vs baseline: 21.8057x; 1.0671x over previous
import functools

import jax
import jax.numpy as jnp
from jax import lax
from jax.experimental import pallas as pl
from jax.experimental.pallas import tpu as pltpu
from jax.experimental.pallas import tpu_sc as plsc

_LANES = 16


def _histogram_sc(ei_flat, n_nodes):
    info = plsc.get_sparse_core_info()
    nc, ns = info.num_cores, info.num_subcores
    nw = nc * ns
    e = ei_flat.shape[0] // 2
    e_per_w = e // nw
    assert e % nw == 0 and e_per_w % _LANES == 0 and n_nodes % _LANES == 0

    mesh = plsc.VectorSubcoreMesh(core_axis_name="c", subcore_axis_name="s")

    @functools.partial(
        pl.kernel,
        mesh=mesh,
        out_type=jax.ShapeDtypeStruct((nw, n_nodes), jnp.float32),
        scratch_types=[
            pltpu.VMEM((e_per_w,), jnp.int32),
            pltpu.VMEM((n_nodes,), jnp.float32),
        ],
        compiler_params=pltpu.CompilerParams(needs_layout_passes=False),
    )
    def hist_kernel(ei_hbm, out_hbm, idx_v, hist_v):
        wid = lax.axis_index("s") * nc + lax.axis_index("c")
        base = e + wid * e_per_w
        pltpu.sync_copy(ei_hbm.at[pl.ds(base, e_per_w)], idx_v)

        zeros = jnp.zeros((_LANES,), jnp.float32)
        unroll = 25
        n_zero = n_nodes // _LANES

        def zero_body(i, carry):
            for j in range(unroll):
                hist_v[pl.ds((i * unroll + j) * _LANES, _LANES)] = zeros
            return carry

        lax.fori_loop(0, n_zero // unroll, zero_body, 0)

        ones = jnp.ones((_LANES,), jnp.float32)
        n_scat = e_per_w // _LANES

        def scat_body(i, carry):
            for j in range(unroll):
                idx = idx_v[pl.ds((i * unroll + j) * _LANES, _LANES)]
                plsc.addupdate_scatter(hist_v, [idx], ones)
            return carry

        lax.fori_loop(0, n_scat // unroll, scat_body, 0)

        pltpu.sync_copy(hist_v, out_hbm.at[wid])

    return hist_kernel(ei_flat)


def _fused_tc(partials, x, w, b2d, inv_e):
    n, c_in = x.shape
    c_out = w.shape[0]

    def body(p_ref, x_ref, w_ref, b_ref, out_ref):
        xv = x_ref[...]
        w1 = w_ref[:, :c_in]
        w2 = w_ref[:, c_in:]
        pm = lax.dot_general(p_ref[...], xv, (((1,), (0,)), ((), ())),
                             preferred_element_type=jnp.float32)
        m = jnp.sum(pm, axis=0, keepdims=True) * inv_e
        m8 = jnp.broadcast_to(m, (8, c_in))
        const8 = lax.dot_general(m8, w2, (((1,), (1,)), ((), ())),
                                 preferred_element_type=jnp.float32)
        const = const8[0:1, :] + b_ref[...]
        y = lax.dot_general(xv, w1, (((1,), (1,)), ((), ())),
                            preferred_element_type=jnp.float32)
        out_ref[...] = y + const

    return pl.pallas_call(
        body,
        out_shape=jax.ShapeDtypeStruct((n, c_out), jnp.float32),
    )(partials, x, w, b2d)


def kernel(x, edge_index, W, b):
    n, _ = x.shape
    e = edge_index.shape[1]
    partials = _histogram_sc(edge_index.astype(jnp.int32).reshape(-1), n)
    return _fused_tc(partials, x, W, b.reshape(1, -1), 1.0 / e)
